# baseline (speedup 1.0000x reference)
import jax
import jax.numpy as jnp
from jax.experimental import pallas as pl
from jax.experimental.pallas import tpu as pltpu


def _body(B, NT, TP, V1, xpt_ref, xg_ref, xgt_ref, mrow_ref, out_ref,
          lp_s):
    g_sum = jnp.float32(0.0)
    for b in range(B):
        xg = xg_ref[b]
        xgt = xgt_ref[b]
        g2r = jnp.sum(xgt * xgt, axis=0, keepdims=True)
        g2c = jnp.transpose(g2r)

        def tile(t, acc):
            xpt = xpt_ref[b, t]
            crosst = jax.lax.dot_general(
                xg, -2.0 * xpt, (((1,), (0,)), ((), ())),
                preferred_element_type=jnp.float32)
            mrow = mrow_ref[b * NT + t]
            p2r = jnp.sum(xpt * xpt, axis=0, keepdims=True)
            pmr = p2r + jnp.where(mrow > 0.0, 0.0, jnp.inf)
            dt = (crosst + g2c) + pmr
            lpmin = jnp.min(dt, axis=0, keepdims=True)
            lp_s[b * NT + t] = jnp.where(
                mrow > 0.0,
                jnp.sqrt(jnp.maximum(lpmin, 0.0)) * 100.0,
                jnp.inf)
            gcol = jnp.min(dt, axis=1, keepdims=True)
            return jnp.minimum(acc, gcol)

        acc = jax.lax.fori_loop(
            0, NT, tile, jnp.full((V1, 1), jnp.inf, jnp.float32),
            unroll=2)
        accr = jnp.transpose(acc)
        g_sum = g_sum + jnp.sum(
            jnp.sqrt(jnp.maximum(accr, 0.0)) * 100.0)

    lp = lp_s[...]
    mv = mrow_ref[...]
    n = jnp.sum(mv)
    idxf = jnp.float32(0.98) * (n - 1.0)
    low = jnp.floor(idxf)
    hw = idxf - low
    lw = 1.0 - hw
    low_i = jnp.clip(low, 0.0, n - 1.0).astype(jnp.int32)
    high_i = jnp.clip(low + 1.0, 0.0, n - 1.0).astype(jnp.int32)

    li = jax.lax.bitcast_convert_type(lp, jnp.int32)

    def sel(k, prefix):
        bit = 30 - k
        one = jnp.int32(1)
        t_mid = prefix + (jax.lax.shift_left(one, bit) - 1)
        cnt = jnp.sum((li <= t_mid).astype(jnp.int32))
        return jnp.where(cnt >= low_i + 1, prefix,
                         prefix + jax.lax.shift_left(one, bit))

    s_low = jax.lax.fori_loop(0, 31, sel, jnp.int32(0))
    s_low_f = jnp.max(jnp.where(li <= s_low, lp, -jnp.inf))
    cnt_le = jnp.sum((li <= s_low).astype(jnp.int32))
    nxt = jnp.min(jnp.where(li > s_low, lp, jnp.inf))
    s_high_f = jnp.where(cnt_le >= high_i + 1, s_low_f, nxt)
    q = s_low_f * lw + s_high_f * hw

    keep = lp <= q
    lp_mean = jnp.sum(jnp.where(keep, lp, 0.0)) / jnp.sum(
        keep.astype(jnp.float32))
    out_ref[...] = jnp.broadcast_to(
        lp_mean + g_sum / jnp.float32(B * V1), (1, 1))


def kernel(x_gt, x_pred, mask):
    B, V1, _ = x_gt.shape
    V2 = x_pred.shape[1]
    TP = 1024
    NT = V2 // TP

    xpt4 = jnp.swapaxes(x_pred.reshape(B, NT, TP, 3), 2, 3)
    xg_t = jnp.swapaxes(x_gt, 1, 2)
    m_row = mask.astype(jnp.float32).reshape(B * NT, 1, TP)

    def fused(*refs):
        _body(B, NT, TP, V1, *refs)

    out = pl.pallas_call(
        fused,
        out_shape=jax.ShapeDtypeStruct((1, 1), jnp.float32),
        scratch_shapes=[
            pltpu.VMEM((B * NT, 1, TP), jnp.float32),
        ],
    )(xpt4, x_gt, xg_t, m_row)
    return out.reshape(())


# speedup vs baseline: 1.0405x; 1.0405x over previous
import jax
import jax.numpy as jnp
from jax.experimental import pallas as pl
from jax.experimental.pallas import tpu as pltpu


def _body(B, NT, TP, V1, xpt_ref, xg_ref, mrow_ref, out_ref, lp_s):
    g_sum = jnp.float32(0.0)
    for b in range(B):
        xg = xg_ref[b]
        g2c = jnp.sum(xg * xg, axis=1, keepdims=True)

        def tile(t, acc):
            xpt = xpt_ref[b, t]
            crosst = jax.lax.dot_general(
                xg, -2.0 * xpt, (((1,), (0,)), ((), ())),
                preferred_element_type=jnp.float32)
            mrow = mrow_ref[b * NT + t]
            p2r = jnp.sum(xpt * xpt, axis=0, keepdims=True)
            pmr = p2r + jnp.where(mrow > 0.0, 0.0, jnp.inf)
            dt = (crosst + g2c) + pmr
            lpmin = jnp.min(dt, axis=0, keepdims=True)
            lp_s[b * NT + t] = jnp.where(
                mrow > 0.0,
                jnp.sqrt(jnp.maximum(lpmin, 0.0)) * 100.0,
                jnp.inf)
            gcol = jnp.min(dt, axis=1, keepdims=True)
            return jnp.minimum(acc, gcol)

        acc = jax.lax.fori_loop(
            0, NT, tile, jnp.full((V1, 1), jnp.inf, jnp.float32),
            unroll=2)
        accr = jnp.transpose(acc)
        g_sum = g_sum + jnp.sum(
            jnp.sqrt(jnp.maximum(accr, 0.0)) * 100.0)

    lp = lp_s[...]
    mv = mrow_ref[...]
    n = jnp.sum(mv)
    idxf = jnp.float32(0.98) * (n - 1.0)
    low = jnp.floor(idxf)
    hw = idxf - low
    lw = 1.0 - hw
    low_i = jnp.clip(low, 0.0, n - 1.0).astype(jnp.int32)
    high_i = jnp.clip(low + 1.0, 0.0, n - 1.0).astype(jnp.int32)

    li = jax.lax.bitcast_convert_type(lp, jnp.int32)

    def sel(k, prefix):
        bit = 30 - k
        one = jnp.int32(1)
        t_mid = prefix + (jax.lax.shift_left(one, bit) - 1)
        cnt = jnp.sum((li <= t_mid).astype(jnp.int32))
        return jnp.where(cnt >= low_i + 1, prefix,
                         prefix + jax.lax.shift_left(one, bit))

    s_low = jax.lax.fori_loop(0, 31, sel, jnp.int32(0))
    s_low_f = jnp.max(jnp.where(li <= s_low, lp, -jnp.inf))
    cnt_le = jnp.sum((li <= s_low).astype(jnp.int32))
    nxt = jnp.min(jnp.where(li > s_low, lp, jnp.inf))
    s_high_f = jnp.where(cnt_le >= high_i + 1, s_low_f, nxt)
    q = s_low_f * lw + s_high_f * hw

    keep = lp <= q
    lp_mean = jnp.sum(jnp.where(keep, lp, 0.0)) / jnp.sum(
        keep.astype(jnp.float32))
    out_ref[...] = jnp.broadcast_to(
        lp_mean + g_sum / jnp.float32(B * V1), (1, 1))


def kernel(x_gt, x_pred, mask):
    B, V1, _ = x_gt.shape
    V2 = x_pred.shape[1]
    TP = 1024
    NT = V2 // TP

    xpt4 = jnp.swapaxes(x_pred.reshape(B, NT, TP, 3), 2, 3)
    m_row = mask.astype(jnp.float32).reshape(B * NT, 1, TP)

    def fused(*refs):
        _body(B, NT, TP, V1, *refs)

    out = pl.pallas_call(
        fused,
        out_shape=jax.ShapeDtypeStruct((1, 1), jnp.float32),
        scratch_shapes=[
            pltpu.VMEM((B * NT, 1, TP), jnp.float32),
        ],
    )(xpt4, x_gt, m_row)
    return out.reshape(())


# speedup vs baseline: 1.1838x; 1.1377x over previous
import jax
import jax.numpy as jnp
from jax.experimental import pallas as pl
from jax.experimental.pallas import tpu as pltpu


def _body(B, NT, TP, V1, xpt_ref, xg_ref, mrow_ref, out_ref, lp_s):
    g_sum = jnp.float32(0.0)
    for b in range(B):
        xg = xg_ref[b]
        g2c = jnp.sum(xg * xg, axis=1, keepdims=True)

        def tile(t, acc):
            xpt = xpt_ref[b, t]
            crosst = jax.lax.dot_general(
                xg, -2.0 * xpt, (((1,), (0,)), ((), ())),
                preferred_element_type=jnp.float32)
            mrow = mrow_ref[b * NT + t]
            p2r = jnp.sum(xpt * xpt, axis=0, keepdims=True)
            pmr = p2r + jnp.where(mrow > 0.0, 0.0, jnp.inf)
            dt = (crosst + g2c) + pmr
            lpmin = jnp.min(dt, axis=0, keepdims=True)
            lp_s[b * NT + t] = jnp.where(
                mrow > 0.0,
                jnp.sqrt(jnp.maximum(lpmin, 0.0)) * 100.0,
                jnp.inf)
            gcol = jnp.min(dt, axis=1, keepdims=True)
            return jnp.minimum(acc, gcol)

        acc = jax.lax.fori_loop(
            0, NT, tile, jnp.full((V1, 1), jnp.inf, jnp.float32),
            unroll=4)
        accr = jnp.transpose(acc)
        g_sum = g_sum + jnp.sum(
            jnp.sqrt(jnp.maximum(accr, 0.0)) * 100.0)

    lp = lp_s[...]
    mv = mrow_ref[...]
    n = jnp.sum(mv)
    idxf = jnp.float32(0.98) * (n - 1.0)
    low = jnp.floor(idxf)
    hw = idxf - low
    lw = 1.0 - hw
    low_i = jnp.clip(low, 0.0, n - 1.0).astype(jnp.int32)
    high_i = jnp.clip(low + 1.0, 0.0, n - 1.0).astype(jnp.int32)

    li = jax.lax.bitcast_convert_type(lp, jnp.int32)

    def sel(k, prefix):
        bit = 30 - k
        one = jnp.int32(1)
        t_mid = prefix + (jax.lax.shift_left(one, bit) - 1)
        cnt = jnp.sum((li <= t_mid).astype(jnp.int32))
        return jnp.where(cnt >= low_i + 1, prefix,
                         prefix + jax.lax.shift_left(one, bit))

    s_low = jax.lax.fori_loop(0, 31, sel, jnp.int32(0))
    s_low_f = jnp.max(jnp.where(li <= s_low, lp, -jnp.inf))
    cnt_le = jnp.sum((li <= s_low).astype(jnp.int32))
    nxt = jnp.min(jnp.where(li > s_low, lp, jnp.inf))
    s_high_f = jnp.where(cnt_le >= high_i + 1, s_low_f, nxt)
    q = s_low_f * lw + s_high_f * hw

    keep = lp <= q
    lp_mean = jnp.sum(jnp.where(keep, lp, 0.0)) / jnp.sum(
        keep.astype(jnp.float32))
    out_ref[...] = jnp.broadcast_to(
        lp_mean + g_sum / jnp.float32(B * V1), (1, 1))


def kernel(x_gt, x_pred, mask):
    B, V1, _ = x_gt.shape
    V2 = x_pred.shape[1]
    TP = 1024
    NT = V2 // TP

    xpt4 = jnp.swapaxes(x_pred.reshape(B, NT, TP, 3), 2, 3)
    m_row = mask.astype(jnp.float32).reshape(B * NT, 1, TP)

    def fused(*refs):
        _body(B, NT, TP, V1, *refs)

    out = pl.pallas_call(
        fused,
        out_shape=jax.ShapeDtypeStruct((1, 1), jnp.float32),
        scratch_shapes=[
            pltpu.VMEM((B * NT, 1, TP), jnp.float32),
        ],
    )(xpt4, x_gt, m_row)
    return out.reshape(())


# speedup vs baseline: 1.1974x; 1.0115x over previous
import jax
import jax.numpy as jnp
from jax.experimental import pallas as pl
from jax.experimental.pallas import tpu as pltpu


def _body(B, NT, TP, V1, xpt_ref, xg_ref, mrow_ref, out_ref, lp_s):
    g_sum = jnp.float32(0.0)
    for b in range(B):
        xg = xg_ref[b]
        g2c = jnp.sum(xg * xg, axis=1, keepdims=True)

        def tile(t, acc):
            xpt = xpt_ref[b, t]
            crosst = jax.lax.dot_general(
                xg, -2.0 * xpt, (((1,), (0,)), ((), ())),
                preferred_element_type=jnp.float32)
            mrow = mrow_ref[b * NT + t]
            p2r = jnp.sum(xpt * xpt, axis=0, keepdims=True)
            pmr = p2r + jnp.where(mrow > 0.0, 0.0, jnp.inf)
            dt = (crosst + g2c) + pmr
            lpmin = jnp.min(dt, axis=0, keepdims=True)
            lp_s[b * NT + t] = jnp.where(
                mrow > 0.0,
                jnp.sqrt(jnp.maximum(lpmin, 0.0)) * 100.0,
                jnp.inf)
            gcol = jnp.min(dt, axis=1, keepdims=True)
            return jnp.minimum(acc, gcol)

        acc = jax.lax.fori_loop(
            0, NT, tile, jnp.full((V1, 1), jnp.inf, jnp.float32),
            unroll=4)
        accr = jnp.transpose(acc)
        g_sum = g_sum + jnp.sum(
            jnp.sqrt(jnp.maximum(accr, 0.0)) * 100.0)

    lp = lp_s[...]
    mv = mrow_ref[...]
    n = jnp.sum(mv)
    idxf = jnp.float32(0.98) * (n - 1.0)
    low = jnp.floor(idxf)
    hw = idxf - low
    lw = 1.0 - hw
    low_i = jnp.clip(low, 0.0, n - 1.0).astype(jnp.int32)
    high_i = jnp.clip(low + 1.0, 0.0, n - 1.0).astype(jnp.int32)

    li = jax.lax.bitcast_convert_type(lp, jnp.int32)

    tgt = low_i + 1
    cnt30 = jnp.sum((li <= jnp.int32((1 << 30) - 1)).astype(jnp.int32))
    prefix = jnp.where(cnt30 >= tgt, jnp.int32(0), jnp.int32(1 << 30))
    for h in range(29, 0, -2):
        step = 1 << (h - 1)
        dd = li - prefix
        c1 = jnp.sum((dd <= jnp.int32(step - 1)).astype(jnp.int32))
        c2 = jnp.sum((dd <= jnp.int32(2 * step - 1)).astype(jnp.int32))
        c3 = jnp.sum((dd <= jnp.int32(3 * step - 1)).astype(jnp.int32))
        add = jnp.where(
            c1 >= tgt, jnp.int32(0),
            jnp.where(c2 >= tgt, jnp.int32(step),
                      jnp.where(c3 >= tgt, jnp.int32(2 * step),
                                jnp.int32(3 * step))))
        prefix = prefix + add
    s_low = prefix
    s_low_f = jnp.max(jnp.where(li <= s_low, lp, -jnp.inf))
    cnt_le = jnp.sum((li <= s_low).astype(jnp.int32))
    nxt = jnp.min(jnp.where(li > s_low, lp, jnp.inf))
    s_high_f = jnp.where(cnt_le >= high_i + 1, s_low_f, nxt)
    q = s_low_f * lw + s_high_f * hw

    keep = lp <= q
    lp_mean = jnp.sum(jnp.where(keep, lp, 0.0)) / jnp.sum(
        keep.astype(jnp.float32))
    out_ref[...] = jnp.broadcast_to(
        lp_mean + g_sum / jnp.float32(B * V1), (1, 1))


def kernel(x_gt, x_pred, mask):
    B, V1, _ = x_gt.shape
    V2 = x_pred.shape[1]
    TP = 1024
    NT = V2 // TP

    xpt4 = jnp.swapaxes(x_pred.reshape(B, NT, TP, 3), 2, 3)
    m_row = mask.astype(jnp.float32).reshape(B * NT, 1, TP)

    def fused(*refs):
        _body(B, NT, TP, V1, *refs)

    out = pl.pallas_call(
        fused,
        out_shape=jax.ShapeDtypeStruct((1, 1), jnp.float32),
        scratch_shapes=[
            pltpu.VMEM((B * NT, 1, TP), jnp.float32),
        ],
    )(xpt4, x_gt, m_row)
    return out.reshape(())


# speedup vs baseline: 1.2743x; 1.0642x over previous
import jax
import jax.numpy as jnp
from jax.experimental import pallas as pl
from jax.experimental.pallas import tpu as pltpu


def _body(B, NT, TP, V1, xpt_ref, xg_ref, mrow_ref, out_ref, lp_s):
    g_sum = jnp.float32(0.0)
    for b in range(B):
        xg = xg_ref[b]
        g2c = jnp.sum(xg * xg, axis=1, keepdims=True)

        acc = jnp.full((V1, 1), jnp.inf, jnp.float32)
        for t in range(NT):
            i = b * NT + t
            xpt = xpt_ref[b, t]
            crosst = jax.lax.dot_general(
                xg, -2.0 * xpt, (((1,), (0,)), ((), ())),
                preferred_element_type=jnp.float32)
            mrow = mrow_ref[i:i + 1, :]
            p2r = jnp.sum(xpt * xpt, axis=0, keepdims=True)
            pmr = p2r + jnp.where(mrow > 0.0, 0.0, jnp.inf)
            dt = (crosst + g2c) + pmr
            lpmin = jnp.min(dt, axis=0, keepdims=True)
            lp_s[i:i + 1, :] = jnp.where(
                mrow > 0.0,
                jnp.sqrt(jnp.maximum(lpmin, 0.0)) * 100.0,
                jnp.inf)
            gcol = jnp.min(dt, axis=1, keepdims=True)
            acc = jnp.minimum(acc, gcol)

        accr = jnp.transpose(acc)
        g_sum = g_sum + jnp.sum(
            jnp.sqrt(jnp.maximum(accr, 0.0)) * 100.0)

    lp = lp_s[...]
    mv = mrow_ref[...]
    n = jnp.sum(mv)
    idxf = jnp.float32(0.98) * (n - 1.0)
    low = jnp.floor(idxf)
    hw = idxf - low
    lw = 1.0 - hw
    low_i = jnp.clip(low, 0.0, n - 1.0).astype(jnp.int32)
    high_i = jnp.clip(low + 1.0, 0.0, n - 1.0).astype(jnp.int32)

    li = jax.lax.bitcast_convert_type(lp, jnp.int32)

    tgt = low_i + 1
    cnt30 = jnp.sum((li <= jnp.int32((1 << 30) - 1)).astype(jnp.int32))
    prefix = jnp.where(cnt30 >= tgt, jnp.int32(0), jnp.int32(1 << 30))
    for h in range(29, 0, -2):
        step = 1 << (h - 1)
        dd = li - prefix
        c1 = jnp.sum((dd <= jnp.int32(step - 1)).astype(jnp.int32))
        c2 = jnp.sum((dd <= jnp.int32(2 * step - 1)).astype(jnp.int32))
        c3 = jnp.sum((dd <= jnp.int32(3 * step - 1)).astype(jnp.int32))
        add = jnp.where(
            c1 >= tgt, jnp.int32(0),
            jnp.where(c2 >= tgt, jnp.int32(step),
                      jnp.where(c3 >= tgt, jnp.int32(2 * step),
                                jnp.int32(3 * step))))
        prefix = prefix + add
    s_low = prefix
    s_low_f = jnp.max(jnp.where(li <= s_low, lp, -jnp.inf))
    cnt_le = jnp.sum((li <= s_low).astype(jnp.int32))
    nxt = jnp.min(jnp.where(li > s_low, lp, jnp.inf))
    s_high_f = jnp.where(cnt_le >= high_i + 1, s_low_f, nxt)
    q = s_low_f * lw + s_high_f * hw

    keep = lp <= q
    lp_mean = jnp.sum(jnp.where(keep, lp, 0.0)) / jnp.sum(
        keep.astype(jnp.float32))
    out_ref[...] = jnp.broadcast_to(
        lp_mean + g_sum / jnp.float32(B * V1), (1, 1))


def kernel(x_gt, x_pred, mask):
    B, V1, _ = x_gt.shape
    V2 = x_pred.shape[1]
    TP = 1024
    NT = V2 // TP

    xpt4 = jnp.swapaxes(x_pred.reshape(B, NT, TP, 3), 2, 3)
    m_row = mask.astype(jnp.float32).reshape(B * NT, TP)

    def fused(*refs):
        _body(B, NT, TP, V1, *refs)

    out = pl.pallas_call(
        fused,
        out_shape=jax.ShapeDtypeStruct((1, 1), jnp.float32),
        scratch_shapes=[
            pltpu.VMEM((B * NT, TP), jnp.float32),
        ],
    )(xpt4, x_gt, m_row)
    return out.reshape(())
